# baseline (speedup 1.0000x reference)
import functools

import jax
import jax.numpy as jnp
from jax.experimental import pallas as pl
from jax.experimental.pallas import tpu as pltpu


def _elu(v):
    return jnp.where(v > 0, v, jnp.exp(jnp.minimum(v, 0.0)) - 1.0)


def _linear_kernel(x_ref, w_ref, b_ref, o_ref):
    o_ref[...] = (
        jnp.dot(x_ref[...], w_ref[...], preferred_element_type=jnp.float32)
        + b_ref[...]
    )


def _spmm_fused_kernel(adj_ref, h_ref, w_ref, b_ref, o_ref):
    acc = jnp.dot(adj_ref[...], h_ref[...], preferred_element_type=jnp.float32)
    t = _elu(acc)
    o_ref[...] = (
        jnp.dot(t, w_ref[...], preferred_element_type=jnp.float32) + b_ref[...]
    )


def _spmm_elu_kernel(adj_ref, h_ref, o_ref):
    acc = jnp.dot(adj_ref[...], h_ref[...], preferred_element_type=jnp.float32)
    o_ref[...] = _elu(acc)


@functools.partial(jax.jit, static_argnames=())
def kernel(x, adjs, W0, b0, W1, b1):
    adj = adjs[0]
    n, nfeat = x.shape
    nhid = W0.shape[1]
    b0r = b0.reshape(1, nhid)
    b1r = b1.reshape(1, nhid)

    h0 = pl.pallas_call(
        _linear_kernel,
        out_shape=jax.ShapeDtypeStruct((n, nhid), jnp.float32),
    )(x, W0, b0r)

    bi = 400
    grid = (n // bi,)

    h1 = pl.pallas_call(
        _spmm_fused_kernel,
        grid=grid,
        in_specs=[
            pl.BlockSpec((bi, n), lambda i: (i, 0)),
            pl.BlockSpec((n, nhid), lambda i: (0, 0)),
            pl.BlockSpec((nhid, nhid), lambda i: (0, 0)),
            pl.BlockSpec((1, nhid), lambda i: (0, 0)),
        ],
        out_specs=pl.BlockSpec((bi, nhid), lambda i: (i, 0)),
        out_shape=jax.ShapeDtypeStruct((n, nhid), jnp.float32),
        compiler_params=pltpu.CompilerParams(
            dimension_semantics=("parallel",),
        ),
    )(adj, h0, W1, b1r)

    out = pl.pallas_call(
        _spmm_elu_kernel,
        grid=grid,
        in_specs=[
            pl.BlockSpec((bi, n), lambda i: (i, 0)),
            pl.BlockSpec((n, nhid), lambda i: (0, 0)),
        ],
        out_specs=pl.BlockSpec((bi, nhid), lambda i: (i, 0)),
        out_shape=jax.ShapeDtypeStruct((n, nhid), jnp.float32),
        compiler_params=pltpu.CompilerParams(
            dimension_semantics=("parallel",),
        ),
    )(adj, h1)

    return out
